# Initial kernel scaffold; baseline (speedup 1.0000x reference)
import jax
import jax.numpy as jnp
from jax.experimental import pallas as pl


def kernel(x_paper, edge_inst_auth, edge_auth_paper, edge_cite, edge_paper_field):
    raise NotImplementedError("write your pallas kernel here")


# speedup vs baseline: 2.0099x; 2.0099x over previous
import jax
import jax.numpy as jnp
from jax.experimental import pallas as pl
from jax.experimental.pallas import tpu as pltpu

N_INST = 10000
N_AUTH = 100000
N_PAPER = 100000
N_FIELD = 50000
E_IA = 400000
E_AP = 1600000
E_PP = 1600000
E_PF = 1600000
D_FEAT = 128

_RB = 400


def _modulate(x, pb_un, cit_un, inst_un):
    def body(x_ref, pb_ref, cit_ref, inst_ref, pb2_ref, o_ref, s_ref):
        i = pl.program_id(0)

        @pl.when(i == 0)
        def _():
            ps = jnp.sum(pb2_ref[...])
            cs = jnp.sum(cit_ref[...])
            ins = jnp.sum(inst_ref[...])
            s_ref[0] = 1.0 / jnp.maximum(ps, 1e-12)
            s_ref[1] = (ins / jnp.maximum(ins, 1e-12)
                        + cs / jnp.maximum(cs, 1e-12))

        o_ref[...] = x_ref[...] * (pb_ref[...] * s_ref[0]) + s_ref[1]

    pb2 = _pad2d(pb_un, 800)
    return pl.pallas_call(
        body,
        grid=(N_PAPER // _RB,),
        in_specs=[
            pl.BlockSpec((_RB, D_FEAT), lambda i: (i, 0)),
            pl.BlockSpec((_RB, 1), lambda i: (i, 0)),
            pl.BlockSpec(cit_un.shape, lambda i: (0, 0)),
            pl.BlockSpec(inst_un.shape, lambda i: (0, 0)),
            pl.BlockSpec(pb2.shape, lambda i: (0, 0)),
        ],
        out_specs=pl.BlockSpec((_RB, D_FEAT), lambda i: (i, 0)),
        out_shape=jax.ShapeDtypeStruct((N_PAPER, D_FEAT), jnp.float32),
        scratch_shapes=[pltpu.SMEM((2,), jnp.float32)],
    )(x, pb_un.reshape(N_PAPER, 1), cit_un, inst_un, pb2)


def _pad2d(v, rows):
    n = v.shape[0]
    return jnp.pad(v, (0, rows * 128 - n)).reshape(rows, 128)


def kernel(x_paper, edge_inst_auth, edge_auth_paper, edge_cite, edge_paper_field):
    f32 = jnp.float32
    ia0, ia1 = edge_inst_auth[0], edge_inst_auth[1]
    ap0, ap1 = edge_auth_paper[0], edge_auth_paper[1]
    pp0, pp1 = edge_cite[0], edge_cite[1]
    pf0 = edge_paper_field[0]

    ones = lambda e: jnp.ones(e.shape[0], f32)
    deg_ia = jax.ops.segment_sum(ones(ia0), ia0, num_segments=N_INST)
    deg_pp = jax.ops.segment_sum(ones(pp0), pp0, num_segments=N_PAPER)
    deg_pf = jax.ops.segment_sum(ones(pf0), pf0, num_segments=N_PAPER)
    deg_ap = jax.ops.segment_sum(ones(ap0), ap0, num_segments=N_AUTH)

    inv = lambda s: jnp.where(s > 0, 1.0 / jnp.maximum(s, 1e-12), 0.0)
    inv_ia, inv_pp, inv_pf, inv_ap = inv(deg_ia), inv(deg_pp), inv(deg_pf), inv(deg_ap)
    inst_prior = deg_ia / float(E_IA)
    cit_prior = deg_pp / float(E_PP)

    paper_like = deg_pf * inv_pf

    t1 = jax.ops.segment_sum(paper_like[pp1], pp0, num_segments=N_PAPER)
    cit_un = cit_prior * (inv_pp * t1)

    t2 = jax.ops.segment_sum(paper_like[ap1], ap0, num_segments=N_AUTH)
    auth_like = inv_ap * t2
    t3 = jax.ops.segment_sum(auth_like[ia1], ia0, num_segments=N_INST)
    inst_un = inst_prior * (inv_ia * t3)

    c_pp = inv_pp * cit_prior
    paper_prior = jax.ops.segment_sum(c_pp[pp0], pp1, num_segments=N_PAPER)
    pb_un = paper_like * paper_prior

    return _modulate(x_paper, pb_un, _pad2d(cit_un, 800), _pad2d(inst_un, 80))


# speedup vs baseline: 130.1584x; 64.7601x over previous
import functools

import jax
import jax.numpy as jnp
from jax import lax
from jax.experimental import pallas as pl
from jax.experimental.pallas import tpu as pltpu
from jax.experimental.pallas import tpu_sc as plsc

N_INST = 10000
N_AUTH = 100000
N_PAPER = 100000
N_FIELD = 50000
E_IA = 400000
E_AP = 1600000
E_PP = 1600000
E_PF = 1600000
D_FEAT = 128

_NC, _NS = 2, 16
_NW = _NC * _NS
_KB = 25
_NBLK_BIG = E_PP // (128 * _KB)
_NBLK_IA = E_IA // (128 * _KB)


def _slc(n):
    return -(-(-(-n // _NS)) // 8) * 8


_SL_P = _slc(N_PAPER)
_SL_I = _slc(N_INST)
_NP_P = _NS * _SL_P
_NP_I = _NS * _SL_I

_MESH = plsc.VectorSubcoreMesh(
    core_axis_name="c", subcore_axis_name="s", num_cores=_NC, num_subcores=_NS)

_f32 = jnp.float32
_i32 = jnp.int32


def _wid():
    c = lax.axis_index("c")
    s = lax.axis_index("s")
    return c, s, s * _NC + c


def _nblk_for(wid, nblk):
    return (nblk - wid + _NW - 1) // _NW


def _hist4(pp0r, ap0r, pf0r, ia0r):
    out_type = [
        jax.ShapeDtypeStruct((_NC, _NP_P), _f32),
        jax.ShapeDtypeStruct((_NC, _NP_P), _f32),
        jax.ShapeDtypeStruct((_NC, _NP_P), _f32),
        jax.ShapeDtypeStruct((_NC, _NP_I), _f32),
    ]
    scratch = [
        pltpu.VMEM((_KB, 128), _i32),
        pltpu.VMEM((128,), _f32),
        pltpu.VMEM_SHARED((_NP_P,), _f32),
        pltpu.VMEM_SHARED((_NP_P,), _f32),
        pltpu.VMEM_SHARED((_NP_P,), _f32),
        pltpu.VMEM_SHARED((_NP_I,), _f32),
        pltpu.SemaphoreType.DMA,
    ]

    @functools.partial(pl.kernel, out_type=out_type, mesh=_MESH,
                       scratch_types=scratch,
                       compiler_params=pltpu.CompilerParams(
                           use_tc_tiling_on_sc=False,
                           needs_layout_passes=False))
    def k(pp0_h, ap0_h, pf0_h, ia0_h, ones_h, zeros_h,
          o_pp, o_ap, o_pf, o_ia, idx_v, ones_v,
          acc_pp, acc_ap, acc_pf, acc_ia, sem):
        c, s, wid = _wid()
        pltpu.sync_copy(ones_h, ones_v)
        pltpu.sync_copy(zeros_h, acc_pp.at[pl.ds(s * _SL_P, _SL_P)])
        pltpu.sync_copy(zeros_h, acc_ap.at[pl.ds(s * _SL_P, _SL_P)])
        pltpu.sync_copy(zeros_h, acc_pf.at[pl.ds(s * _SL_P, _SL_P)])
        pltpu.sync_copy(zeros_h.at[pl.ds(0, _SL_I)],
                        acc_ia.at[pl.ds(s * _SL_I, _SL_I)])
        plsc.subcore_barrier()

        def run(e_h, nblk, acc):
            def body(i, carry):
                blk = wid + i * _NW
                pltpu.sync_copy(e_h.at[pl.ds(blk * _KB, _KB)], idx_v)
                cps = [pltpu.async_copy(ones_v, acc.at[idx_v.at[j]], sem,
                                        add=True)
                       for j in range(_KB)]
                for cp in cps:
                    cp.wait()
                return carry
            lax.fori_loop(0, _nblk_for(wid, nblk), body, 0)

        run(pp0_h, _NBLK_BIG, acc_pp)
        run(ap0_h, _NBLK_BIG, acc_ap)
        run(pf0_h, _NBLK_BIG, acc_pf)
        run(ia0_h, _NBLK_IA, acc_ia)
        plsc.subcore_barrier()
        sl = pl.ds(s * _SL_P, _SL_P)
        pltpu.sync_copy(acc_pp.at[sl], o_pp.at[c, sl])
        pltpu.sync_copy(acc_ap.at[sl], o_ap.at[c, sl])
        pltpu.sync_copy(acc_pf.at[sl], o_pf.at[c, sl])
        sli = pl.ds(s * _SL_I, _SL_I)
        pltpu.sync_copy(acc_ia.at[sli], o_ia.at[c, sli])

    ones = jnp.ones((128,), _f32)
    zeros = jnp.zeros((_SL_P,), _f32)
    return k(pp0r, ap0r, pf0r, ia0r, ones, zeros)


def _gseg(table, pairs):
    out_type = [jax.ShapeDtypeStruct((_NC, npad), _f32)
                for (_, _, _, npad) in pairs]
    scratch = ([
        pltpu.VMEM((_NP_P,), _f32),
        pltpu.VMEM((_KB, 128), _i32),
        pltpu.VMEM((_KB, 128), _i32),
        pltpu.VMEM((_KB, 128), _f32),
        pltpu.SemaphoreType.DMA,
    ] + [pltpu.VMEM_SHARED((npad,), _f32) for (_, _, _, npad) in pairs])

    npairs = len(pairs)

    @functools.partial(pl.kernel, out_type=out_type, mesh=_MESH,
                       scratch_types=scratch,
                       compiler_params=pltpu.CompilerParams(
                           use_tc_tiling_on_sc=False,
                           needs_layout_passes=False))
    def k(table_h, *rest):
        idx_hs = rest[:2 * npairs]
        zeros_h = rest[2 * npairs]
        outs = rest[2 * npairs + 1: 2 * npairs + 1 + npairs]
        table_v, gidx_v, sidx_v, vals_v, sem = rest[2 * npairs + 1 + npairs:
                                                    2 * npairs + 1 + npairs + 5]
        accs = rest[2 * npairs + 1 + npairs + 5:]
        c, s, wid = _wid()
        pltpu.sync_copy(table_h, table_v)
        for (_, _, _, npad), acc in zip(pairs, accs):
            sl = npad // _NS
            pltpu.sync_copy(zeros_h.at[pl.ds(0, sl)],
                            acc.at[pl.ds(s * sl, sl)])
        plsc.subcore_barrier()

        for pi, ((_, _, nblk, npad), acc) in enumerate(zip(pairs, accs)):
            g_h, s_h = idx_hs[2 * pi], idx_hs[2 * pi + 1]

            def body(i, carry):
                blk = wid + i * _NW
                pltpu.sync_copy(g_h.at[pl.ds(blk * _KB, _KB)], gidx_v)
                pltpu.sync_copy(s_h.at[pl.ds(blk * _KB, _KB)], sidx_v)
                cps = []
                for j in range(_KB):
                    for i2 in range(8):
                        sl16 = pl.ds(i2 * 16, 16)
                        idx16 = gidx_v[j, sl16]
                        vals_v[j, sl16] = plsc.load_gather(table_v, [idx16])
                    cps.append(pltpu.async_copy(
                        vals_v.at[j], acc.at[sidx_v.at[j]], sem, add=True))
                for cp in cps:
                    cp.wait()
                return carry
            lax.fori_loop(0, _nblk_for(wid, nblk), body, 0)

        plsc.subcore_barrier()
        for (_, _, _, npad), acc, o in zip(pairs, accs, outs):
            sl = npad // _NS
            sld = pl.ds(s * sl, sl)
            pltpu.sync_copy(acc.at[sld], o.at[c, sld])

    zeros = jnp.zeros((_SL_P,), _f32)
    idx_args = []
    for (g, sidx, _, _) in pairs:
        idx_args += [g, sidx]
    return k(table, *idx_args, zeros)


_RB = 400


def _modulate(x, pb_un, cit_un, inst_un):
    def body(x_ref, pb_ref, cit_ref, inst_ref, pb2_ref, o_ref, s_ref):
        i = pl.program_id(0)

        @pl.when(i == 0)
        def _():
            ps = jnp.sum(pb2_ref[...])
            cs = jnp.sum(cit_ref[...])
            ins = jnp.sum(inst_ref[...])
            s_ref[0] = 1.0 / jnp.maximum(ps, 1e-12)
            s_ref[1] = (ins / jnp.maximum(ins, 1e-12)
                        + cs / jnp.maximum(cs, 1e-12))

        o_ref[...] = x_ref[...] * (pb_ref[...] * s_ref[0]) + s_ref[1]

    pb2 = _pad2d(pb_un, 800)
    return pl.pallas_call(
        body,
        grid=(N_PAPER // _RB,),
        in_specs=[
            pl.BlockSpec((_RB, D_FEAT), lambda i: (i, 0)),
            pl.BlockSpec((_RB, 1), lambda i: (i, 0)),
            pl.BlockSpec(cit_un.shape, lambda i: (0, 0)),
            pl.BlockSpec(inst_un.shape, lambda i: (0, 0)),
            pl.BlockSpec(pb2.shape, lambda i: (0, 0)),
        ],
        out_specs=pl.BlockSpec((_RB, D_FEAT), lambda i: (i, 0)),
        out_shape=jax.ShapeDtypeStruct((N_PAPER, D_FEAT), jnp.float32),
        scratch_shapes=[pltpu.SMEM((2,), jnp.float32)],
    )(x, pb_un.reshape(N_PAPER, 1), cit_un, inst_un, pb2)


def _pad2d(v, rows):
    n = v.shape[0]
    return jnp.pad(v, (0, rows * 128 - n)).reshape(rows, 128)


def _padto(v, n):
    return jnp.pad(v, (0, n - v.shape[0]))


def kernel(x_paper, edge_inst_auth, edge_auth_paper, edge_cite, edge_paper_field):
    ia0, ia1 = edge_inst_auth[0], edge_inst_auth[1]
    ap0, ap1 = edge_auth_paper[0], edge_auth_paper[1]
    pp0, pp1 = edge_cite[0], edge_cite[1]
    pf0 = edge_paper_field[0]

    r = lambda e: e.reshape(-1, 128)
    pp0r, pp1r, ap0r, ap1r, pf0r = r(pp0), r(pp1), r(ap0), r(ap1), r(pf0)
    ia0r, ia1r = r(ia0), r(ia1)

    h_pp, h_ap, h_pf, h_ia = _hist4(pp0r, ap0r, pf0r, ia0r)
    deg_pp = (h_pp[0] + h_pp[1])[:N_PAPER]
    deg_ap = (h_ap[0] + h_ap[1])[:N_AUTH]
    deg_pf = (h_pf[0] + h_pf[1])[:N_PAPER]
    deg_ia = (h_ia[0] + h_ia[1])[:N_INST]

    inv = lambda d: jnp.where(d > 0, 1.0 / jnp.maximum(d, 1e-12), 0.0)
    inv_ia, inv_pp, inv_pf, inv_ap = inv(deg_ia), inv(deg_pp), inv(deg_pf), inv(deg_ap)
    inst_prior = deg_ia / float(E_IA)
    cit_prior = deg_pp / float(E_PP)
    paper_like = deg_pf * inv_pf
    c_pp = inv_pp * cit_prior

    p_t1, p_t2 = _gseg(_padto(paper_like, _NP_P), [
        (pp1r, pp0r, _NBLK_BIG, _NP_P),
        (ap1r, ap0r, _NBLK_BIG, _NP_P),
    ])
    t1 = (p_t1[0] + p_t1[1])[:N_PAPER]
    t2 = (p_t2[0] + p_t2[1])[:N_AUTH]
    cit_un = cit_prior * (inv_pp * t1)
    auth_like = inv_ap * t2

    (p_pp,) = _gseg(_padto(c_pp, _NP_P), [(pp0r, pp1r, _NBLK_BIG, _NP_P)])
    paper_prior = (p_pp[0] + p_pp[1])[:N_PAPER]

    (p_t3,) = _gseg(_padto(auth_like, _NP_P), [(ia1r, ia0r, _NBLK_IA, _NP_I)])
    t3 = (p_t3[0] + p_t3[1])[:N_INST]
    inst_un = inst_prior * (inv_ia * t3)

    pb_un = paper_like * paper_prior

    return _modulate(x_paper, pb_un, _pad2d(cit_un, 800), _pad2d(inst_un, 80))


# speedup vs baseline: 202.5754x; 1.5564x over previous
import functools

import jax
import jax.numpy as jnp
from jax import lax
from jax.experimental import pallas as pl
from jax.experimental.pallas import tpu as pltpu
from jax.experimental.pallas import tpu_sc as plsc

N_INST = 10000
N_AUTH = 100000
N_PAPER = 100000
N_FIELD = 50000
E_IA = 400000
E_AP = 1600000
E_PP = 1600000
E_PF = 1600000
D_FEAT = 128

_NC, _NS = 2, 16
_NW = _NC * _NS
_KB = 25
_NBLK_BIG = E_PP // (128 * _KB)
_NBLK_IA = E_IA // (128 * _KB)

_SL_P = 6256
_NP_P = _NS * _SL_P
_NBW = 3200

_MESH = plsc.VectorSubcoreMesh(
    core_axis_name="c", subcore_axis_name="s", num_cores=_NC, num_subcores=_NS)
_SC_PARAMS = pltpu.CompilerParams(
    use_tc_tiling_on_sc=False, needs_layout_passes=False)

_f32 = jnp.float32
_i32 = jnp.int32


def _wid():
    c = lax.axis_index("c")
    s = lax.axis_index("s")
    return c, s, s * _NC + c


def _nblk_for(wid, nblk):
    return (nblk - wid + _NW - 1) // _NW


def _hist3(pf0r, pp1r, ia1r):
    out_type = [jax.ShapeDtypeStruct((_NC, _NP_P), _f32) for _ in range(3)]
    scratch = [
        pltpu.VMEM((_KB, 128), _i32),
        pltpu.VMEM((128,), _f32),
        pltpu.VMEM_SHARED((_NP_P,), _f32),
        pltpu.VMEM_SHARED((_NP_P,), _f32),
        pltpu.VMEM_SHARED((_NP_P,), _f32),
        pltpu.SemaphoreType.DMA,
    ]

    @functools.partial(pl.kernel, out_type=out_type, mesh=_MESH,
                       scratch_types=scratch, compiler_params=_SC_PARAMS)
    def k(pf_h, pp_h, ia_h, ones_h, zeros_h,
          o_pf, o_pp, o_ia, idx_v, ones_v, acc_pf, acc_pp, acc_ia, sem):
        c, s, wid = _wid()
        sl = pl.ds(s * _SL_P, _SL_P)
        pltpu.sync_copy(ones_h, ones_v)
        pltpu.sync_copy(zeros_h, acc_pf.at[sl])
        pltpu.sync_copy(zeros_h, acc_pp.at[sl])
        pltpu.sync_copy(zeros_h, acc_ia.at[sl])
        plsc.subcore_barrier()

        def run(e_h, nblk, acc):
            def body(i, carry):
                blk = wid + i * _NW
                pltpu.sync_copy(e_h.at[pl.ds(blk * _KB, _KB)], idx_v)
                cps = [pltpu.async_copy(ones_v, acc.at[idx_v.at[j]], sem,
                                        add=True)
                       for j in range(_KB)]
                for cp in cps:
                    cp.wait()
                return carry
            lax.fori_loop(0, _nblk_for(wid, nblk), body, 0)

        run(pf_h, _NBLK_BIG, acc_pf)
        run(pp_h, _NBLK_BIG, acc_pp)
        run(ia_h, _NBLK_IA, acc_ia)
        plsc.subcore_barrier()
        pltpu.sync_copy(acc_pf.at[sl], o_pf.at[c, sl])
        pltpu.sync_copy(acc_pp.at[sl], o_pp.at[c, sl])
        pltpu.sync_copy(acc_ia.at[sl], o_ia.at[c, sl])

    ones = jnp.ones((128,), _f32)
    zeros = jnp.zeros((_SL_P,), _f32)
    return k(pf0r, pp1r, ia1r, ones, zeros)


def _reach(ap0r, ap1r, ahbits, plbits):
    out_type = [jax.ShapeDtypeStruct((_NC, _NS, 16), _f32)]
    scratch = [
        pltpu.VMEM((_NBW,), _i32),
        pltpu.VMEM((_NBW,), _i32),
        pltpu.VMEM((_KB, 128), _i32),
        pltpu.VMEM((_KB, 128), _i32),
        pltpu.VMEM((16,), _f32),
    ]

    @functools.partial(pl.kernel, out_type=out_type, mesh=_MESH,
                       scratch_types=scratch, compiler_params=_SC_PARAMS)
    def k(ap0_h, ap1_h, ahb_h, plb_h, o, ahb_v, plb_v, a_v, p_v, out_v):
        c, s, wid = _wid()
        pltpu.sync_copy(ahb_h, ahb_v)
        pltpu.sync_copy(plb_h, plb_v)

        def bit_at(tab, idx16):
            w = plsc.load_gather(tab, [lax.shift_right_logical(idx16, 5)])
            return lax.shift_right_logical(w, idx16 & 31) & 1

        def body(i, acc):
            blk = wid + i * _NW
            pltpu.sync_copy(ap0_h.at[pl.ds(blk * _KB, _KB)], a_v)
            pltpu.sync_copy(ap1_h.at[pl.ds(blk * _KB, _KB)], p_v)
            for j in range(_KB):
                for i2 in range(8):
                    sl16 = pl.ds(i2 * 16, 16)
                    hit = bit_at(ahb_v, a_v[j, sl16]) & bit_at(plb_v, p_v[j, sl16])
                    acc = acc + hit.astype(_f32)
            return acc

        acc = lax.fori_loop(0, _nblk_for(wid, _NBLK_BIG), body,
                            jnp.zeros((16,), _f32))
        out_v[...] = acc
        pltpu.sync_copy(out_v, o.at[c, s])

    return k(ap0r, ap1r, ahbits, plbits)[0]


def _pack_bits(pos):
    b = jnp.pad(pos.astype(_i32), (0, _NBW * 32 - _NP_P)).reshape(_NBW, 32)
    k = jnp.arange(16, dtype=_i32)
    low = jnp.sum(b[:, :16] << k, axis=1)
    high = jnp.sum(b[:, 16:] << k, axis=1)
    return low | (high << 16)


_RB = 400


def _pad2d(v, rows):
    n = v.shape[0]
    return jnp.pad(v, (0, rows * 128 - n)).reshape(rows, 128)


def _modulate(x, pb_un, reach_part):
    def body(x_ref, pb_ref, pb2_ref, rch_ref, o_ref, s_ref):
        i = pl.program_id(0)

        @pl.when(i == 0)
        def _():
            ps = jnp.sum(pb2_ref[...])
            rs = jnp.sum(rch_ref[...])
            s_ref[0] = 1.0 / jnp.maximum(ps, 1e-12)
            s_ref[1] = (jnp.where(ps > 0, 1.0, 0.0)
                        + jnp.where(rs > 0, 1.0, 0.0))

        o_ref[...] = x_ref[...] * (pb_ref[...] * s_ref[0]) + s_ref[1]

    pb2 = _pad2d(pb_un, 800)
    rch = reach_part.reshape(_NC * _NS, 16)
    return pl.pallas_call(
        body,
        grid=(N_PAPER // _RB,),
        in_specs=[
            pl.BlockSpec((_RB, D_FEAT), lambda i: (i, 0)),
            pl.BlockSpec((_RB, 1), lambda i: (i, 0)),
            pl.BlockSpec(pb2.shape, lambda i: (0, 0)),
            pl.BlockSpec(rch.shape, lambda i: (0, 0)),
        ],
        out_specs=pl.BlockSpec((_RB, D_FEAT), lambda i: (i, 0)),
        out_shape=jax.ShapeDtypeStruct((N_PAPER, D_FEAT), jnp.float32),
        scratch_shapes=[pltpu.SMEM((2,), jnp.float32)],
    )(x, pb_un.reshape(N_PAPER, 1), pb2, rch)


def kernel(x_paper, edge_inst_auth, edge_auth_paper, edge_cite, edge_paper_field):
    r = lambda e: e.reshape(-1, 128)
    pf0r = r(edge_paper_field[0])
    pp1r = r(edge_cite[1])
    ia1r = r(edge_inst_auth[1])
    ap0r, ap1r = r(edge_auth_paper[0]), r(edge_auth_paper[1])

    h_pf, h_pp1, h_ia1 = _hist3(pf0r, pp1r, ia1r)
    deg_pf = h_pf[0] + h_pf[1]
    indeg_pp = h_pp1[0] + h_pp1[1]
    ahi = h_ia1[0] + h_ia1[1]

    pf_pos = deg_pf > 0
    pb_un = (jnp.where(pf_pos, 1.0, 0.0) * indeg_pp)[:N_PAPER]

    reach_part = _reach(ap0r, ap1r, _pack_bits(ahi > 0), _pack_bits(pf_pos))

    return _modulate(x_paper, pb_un, reach_part)
